# Initial kernel scaffold; baseline (speedup 1.0000x reference)
import functools

import jax
import jax.numpy as jnp
from jax import lax
from jax.experimental import pallas as pl
from jax.experimental.pallas import tpu as pltpu, tpu_sc as plsc

NC = 2
NS = 16
NW = NC * NS
L = 16


def _feat_body(x_ref, w_ref, al_ref, ar_ref, f_ref, el_ref, er_ref):
    f = jnp.dot(x_ref[...], w_ref[...], preferred_element_type=jnp.float32)
    f_ref[...] = f
    el_ref[...] = jnp.sum(f * al_ref[...], axis=1, keepdims=True)
    er_ref[...] = jnp.sum(f * ar_ref[...], axis=1, keepdims=True)


def _tc_feat(in_feat, W, attn_l, attn_r):
    n, d = in_feat.shape
    h = W.shape[1]
    blk = 1000
    grid = n // blk
    feat, el, er = pl.pallas_call(
        _feat_body,
        grid=(grid,),
        in_specs=[
            pl.BlockSpec((blk, d), lambda i: (i, 0)),
            pl.BlockSpec((d, h), lambda i: (0, 0)),
            pl.BlockSpec((1, h), lambda i: (0, 0)),
            pl.BlockSpec((1, h), lambda i: (0, 0)),
        ],
        out_specs=[
            pl.BlockSpec((blk, h), lambda i: (i, 0)),
            pl.BlockSpec((blk, 1), lambda i: (i, 0)),
            pl.BlockSpec((blk, 1), lambda i: (i, 0)),
        ],
        out_shape=[
            jax.ShapeDtypeStruct((n, h), jnp.float32),
            jax.ShapeDtypeStruct((n, 1), jnp.float32),
            jax.ShapeDtypeStruct((n, 1), jnp.float32),
        ],
    )(in_feat, W, attn_l.reshape(1, h), attn_r.reshape(1, h))
    return feat, el.reshape(n), er.reshape(n)


def _sc_edge_pass(src2d, dst2d, feat, el, er, n, e_total, h):
    rows_per_tile = src2d.shape[0] // NW
    blocks_per_tile = rows_per_tile // 8
    nrows_tile = n // NS
    assert n % (NS * 5) == 0
    zchunk = nrows_tile // 5

    mesh = plsc.VectorSubcoreMesh(core_axis_name="c", subcore_axis_name="s")

    @functools.partial(
        pl.kernel,
        mesh=mesh,
        out_type=[
            jax.ShapeDtypeStruct((NC, n, h), jnp.float32),
            jax.ShapeDtypeStruct((NW, n), jnp.float32),
        ],
        scratch_types=[
            pltpu.VMEM((n,), jnp.float32),
            pltpu.VMEM((n,), jnp.float32),
            pltpu.VMEM((n,), jnp.float32),
            pltpu.VMEM((8, 128), jnp.int32),
            pltpu.VMEM((8, 128), jnp.int32),
            pltpu.VMEM((8, 128), jnp.float32),
            pltpu.VMEM((128, 128), jnp.float32),
            pltpu.VMEM_SHARED((10000, 128), jnp.float32),
            pltpu.SemaphoreType.DMA,
        ],
    )
    def edge_kernel(src_r, dst_r, feat_r, el_r, er_r, h_out, den_out,
                    el_v, er_v, den_v, src_v, dst_v, ex_v, rows_v, h_sh, sem):
        cid = lax.axis_index("c")
        sid = lax.axis_index("s")
        wid = sid * NC + cid

        def zden(i, c):
            den_v[pl.ds(i * L, L)] = jnp.zeros((L,), jnp.float32)
            return c
        lax.fori_loop(0, n // L, zden, 0)

        def zrow(i, c):
            for k in range(h // L):
                rows_v[i, pl.ds(k * L, L)] = jnp.zeros((L,), jnp.float32)
            return c
        lax.fori_loop(0, zchunk, zrow, 0)
        for k in range(5):
            pltpu.sync_copy(rows_v.at[pl.ds(0, zchunk)],
                            h_sh.at[pl.ds(sid * nrows_tile + k * zchunk, zchunk)])

        pltpu.sync_copy(el_r, el_v)
        pltpu.sync_copy(er_r, er_v)

        plsc.subcore_barrier()

        row_base = wid * rows_per_tile

        def block(jj, c):
            row0 = row_base + jj * 8
            pltpu.sync_copy(src_r.at[pl.ds(row0, 8)], src_v)
            pltpu.sync_copy(dst_r.at[pl.ds(row0, 8)], dst_v)

            def cex(i, cc):
                g = i // 8
                c16 = i % 8
                s = src_v[g, pl.ds(c16 * L, L)]
                d = dst_v[g, pl.ds(c16 * L, L)]
                ev = plsc.load_gather(el_v, [s]) + plsc.load_gather(er_v, [d])
                ev = jnp.where(ev >= 0, ev, ev * 0.2)
                ex = jnp.exp(ev)
                eid = (row0 + g) * 128 + c16 * L + lax.broadcasted_iota(jnp.int32, (L,), 0)
                ex = jnp.where(eid < e_total, ex, 0.0)
                ex_v[g, pl.ds(c16 * L, L)] = ex
                plsc.addupdate_scatter(den_v, [d], ex)
                return cc
            lax.fori_loop(0, 64, cex, 0)

            for g in range(8):
                pltpu.async_copy(feat_r.at[src_v.at[g]], rows_v, sem).wait()

                def scale(r, cc):
                    sc = plsc.load_gather(
                        ex_v, [jnp.full((L,), g, jnp.int32), jnp.full((L,), r, jnp.int32)])
                    for k in range(h // L):
                        rows_v[r, pl.ds(k * L, L)] = rows_v[r, pl.ds(k * L, L)] * sc
                    return cc
                lax.fori_loop(0, 128, scale, 0)

                pltpu.sync_copy(rows_v, h_sh.at[dst_v.at[g]], add=True)
            return c
        lax.fori_loop(0, blocks_per_tile, block, 0)

        plsc.subcore_barrier()

        pltpu.sync_copy(den_v, den_out.at[wid])
        for k in range(5):
            sl = pl.ds(sid * nrows_tile + k * zchunk, zchunk)
            pltpu.sync_copy(h_sh.at[sl], h_out.at[cid, sl])

    return edge_kernel(src2d, dst2d, feat, el, er)


def _final_body(h_ref, den_ref, bias_ref, w2_ref, b2_ref, out_ref):
    ht = h_ref[0] + h_ref[1]
    dt = jnp.sum(den_ref[...], axis=0)[:, None]
    hh = ht / jnp.maximum(dt, 1e-9)
    hh = jnp.maximum(hh + bias_ref[...], 0.0)
    logits = jnp.dot(hh, w2_ref[...], preferred_element_type=jnp.float32) + b2_ref[...]
    out_ref[...] = jax.nn.sigmoid(logits)


def _tc_final(h_part, den_part, bias, W2, b2):
    _, n, h = h_part.shape
    c = W2.shape[1]
    blk = 1000
    grid = n // blk
    return pl.pallas_call(
        _final_body,
        grid=(grid,),
        in_specs=[
            pl.BlockSpec((NC, blk, h), lambda i: (0, i, 0)),
            pl.BlockSpec((NW, blk), lambda i: (0, i)),
            pl.BlockSpec((1, h), lambda i: (0, 0)),
            pl.BlockSpec((h, c), lambda i: (0, 0)),
            pl.BlockSpec((1, c), lambda i: (0, 0)),
        ],
        out_specs=pl.BlockSpec((blk, c), lambda i: (i, 0)),
        out_shape=jax.ShapeDtypeStruct((n, c), jnp.float32),
    )(h_part, den_part, bias.reshape(1, h), W2, b2.reshape(1, c))


def kernel(edge_index, in_feat, W, attn_l, attn_r, bias, W2, b2):
    n, _ = in_feat.shape
    h = W.shape[1]
    e_total = edge_index.shape[1]

    ept = -(-e_total // (NW * 1024)) * 1024
    epad = NW * ept
    src = edge_index[0].astype(jnp.int32)
    dst = edge_index[1].astype(jnp.int32)
    src = jnp.pad(src, (0, epad - e_total)).reshape(epad // 128, 128)
    dst = jnp.pad(dst, (0, epad - e_total)).reshape(epad // 128, 128)

    feat, el, er = _tc_feat(in_feat, W, attn_l, attn_r)
    h_part, den_part = _sc_edge_pass(src, dst, feat, el, er, n, e_total, h)
    return _tc_final(h_part, den_part, bias, W2, b2)


# speedup vs baseline: 15.3627x; 15.3627x over previous
import functools

import jax
import jax.numpy as jnp
from jax import lax
from jax.experimental import pallas as pl
from jax.experimental.pallas import tpu as pltpu, tpu_sc as plsc

NC = 2
NS = 16
NW = NC * NS
L = 16


def _feat_body(x_ref, w_ref, al_ref, ar_ref, f_ref, el_ref, er_ref):
    f = jnp.dot(x_ref[...], w_ref[...], preferred_element_type=jnp.float32)
    f_ref[...] = f
    el_ref[...] = jnp.sum(f * al_ref[...], axis=1, keepdims=True)
    er_ref[...] = jnp.sum(f * ar_ref[...], axis=1, keepdims=True)


def _tc_feat(in_feat, W, attn_l, attn_r):
    n, d = in_feat.shape
    h = W.shape[1]
    blk = 1000
    grid = n // blk
    feat, el, er = pl.pallas_call(
        _feat_body,
        grid=(grid,),
        in_specs=[
            pl.BlockSpec((blk, d), lambda i: (i, 0)),
            pl.BlockSpec((d, h), lambda i: (0, 0)),
            pl.BlockSpec((1, h), lambda i: (0, 0)),
            pl.BlockSpec((1, h), lambda i: (0, 0)),
        ],
        out_specs=[
            pl.BlockSpec((blk, h), lambda i: (i, 0)),
            pl.BlockSpec((blk, 1), lambda i: (i, 0)),
            pl.BlockSpec((blk, 1), lambda i: (i, 0)),
        ],
        out_shape=[
            jax.ShapeDtypeStruct((n, h), jnp.float32),
            jax.ShapeDtypeStruct((n, 1), jnp.float32),
            jax.ShapeDtypeStruct((n, 1), jnp.float32),
        ],
    )(in_feat, W, attn_l.reshape(1, h), attn_r.reshape(1, h))
    return feat, el.reshape(n), er.reshape(n)


def _sc_edge_pass(src2d, dst2d, feat, el, er, n, e_total, h):
    rows_per_tile = src2d.shape[0] // NW
    blocks_per_tile = rows_per_tile // 8
    nrows_tile = n // NS
    assert n % (NS * 5) == 0
    zchunk = nrows_tile // 5

    mesh = plsc.VectorSubcoreMesh(core_axis_name="c", subcore_axis_name="s")

    @functools.partial(
        pl.kernel,
        mesh=mesh,
        compiler_params=pltpu.CompilerParams(use_tc_tiling_on_sc=False,
                                              needs_layout_passes=False),
        out_type=[
            jax.ShapeDtypeStruct((NC, n, h), jnp.float32),
            jax.ShapeDtypeStruct((NW, n), jnp.float32),
        ],
        scratch_types=[
            pltpu.VMEM((n,), jnp.float32),
            pltpu.VMEM((n,), jnp.float32),
            pltpu.VMEM((n,), jnp.float32),
            pltpu.VMEM((8, 128), jnp.int32),
            pltpu.VMEM((8, 128), jnp.int32),
            pltpu.VMEM((8, 128), jnp.float32),
            pltpu.VMEM((128, 128), jnp.float32),
            pltpu.VMEM_SHARED((10000, 128), jnp.float32),
            pltpu.SemaphoreType.DMA,
        ],
    )
    def edge_kernel(src_r, dst_r, feat_r, el_r, er_r, h_out, den_out,
                    el_v, er_v, den_v, src_v, dst_v, ex_v, rows_v, h_sh, sem):
        cid = lax.axis_index("c")
        sid = lax.axis_index("s")
        wid = sid * NC + cid

        def zden(i, c):
            den_v[pl.ds(i * L, L)] = jnp.zeros((L,), jnp.float32)
            return c
        lax.fori_loop(0, n // L, zden, 0)

        def zrow(i, c):
            for k in range(h // L):
                rows_v[i, pl.ds(k * L, L)] = jnp.zeros((L,), jnp.float32)
            return c
        lax.fori_loop(0, zchunk, zrow, 0)
        for k in range(5):
            pltpu.sync_copy(rows_v.at[pl.ds(0, zchunk)],
                            h_sh.at[pl.ds(sid * nrows_tile + k * zchunk, zchunk)])

        pltpu.sync_copy(el_r, el_v)
        pltpu.sync_copy(er_r, er_v)

        plsc.subcore_barrier()

        row_base = wid * rows_per_tile

        def block(jj, c):
            row0 = row_base + jj * 8
            pltpu.sync_copy(src_r.at[pl.ds(row0, 8)], src_v)
            pltpu.sync_copy(dst_r.at[pl.ds(row0, 8)], dst_v)

            def cex(i, cc):
                g = i // 8
                c16 = i % 8
                s = src_v[g, pl.ds(c16 * L, L)]
                d = dst_v[g, pl.ds(c16 * L, L)]
                ev = plsc.load_gather(el_v, [s]) + plsc.load_gather(er_v, [d])
                ev = jnp.where(ev >= 0, ev, ev * 0.2)
                ex = jnp.exp(ev)
                eid = (row0 + g) * 128 + c16 * L + lax.broadcasted_iota(jnp.int32, (L,), 0)
                ex = jnp.where(eid < e_total, ex, 0.0)
                ex_v[g, pl.ds(c16 * L, L)] = ex
                plsc.addupdate_scatter(den_v, [d], ex)
                return cc
            lax.fori_loop(0, 64, cex, 0)

            for g in range(8):
                pltpu.async_copy(feat_r.at[src_v.at[g]], rows_v, sem).wait()

                def scale(r, cc):
                    sc = plsc.load_gather(
                        ex_v, [jnp.full((L,), g, jnp.int32), jnp.full((L,), r, jnp.int32)])
                    for k in range(h // L):
                        rows_v[r, pl.ds(k * L, L)] = rows_v[r, pl.ds(k * L, L)] * sc
                    return cc
                lax.fori_loop(0, 128, scale, 0)

                pltpu.sync_copy(rows_v, h_sh.at[dst_v.at[g]], add=True)
            return c
        lax.fori_loop(0, blocks_per_tile, block, 0)

        plsc.subcore_barrier()

        pltpu.sync_copy(den_v, den_out.at[wid])
        for k in range(5):
            sl = pl.ds(sid * nrows_tile + k * zchunk, zchunk)
            pltpu.sync_copy(h_sh.at[sl], h_out.at[cid, sl])

    return edge_kernel(src2d, dst2d, feat, el, er)


def _final_body(h_ref, den_ref, bias_ref, w2_ref, b2_ref, out_ref):
    ht = h_ref[0] + h_ref[1]
    dt = jnp.sum(den_ref[...], axis=0)[:, None]
    hh = ht / jnp.maximum(dt, 1e-9)
    hh = jnp.maximum(hh + bias_ref[...], 0.0)
    logits = jnp.dot(hh, w2_ref[...], preferred_element_type=jnp.float32) + b2_ref[...]
    out_ref[...] = jax.nn.sigmoid(logits)


def _tc_final(h_part, den_part, bias, W2, b2):
    _, n, h = h_part.shape
    c = W2.shape[1]
    return pl.pallas_call(
        _final_body,
        out_shape=jax.ShapeDtypeStruct((n, c), jnp.float32),
    )(h_part, den_part, bias.reshape(1, h), W2, b2.reshape(1, c))


def kernel(edge_index, in_feat, W, attn_l, attn_r, bias, W2, b2):
    n, _ = in_feat.shape
    h = W.shape[1]
    e_total = edge_index.shape[1]

    ept = -(-e_total // (NW * 1024)) * 1024
    epad = NW * ept
    src = edge_index[0].astype(jnp.int32)
    dst = edge_index[1].astype(jnp.int32)
    src = jnp.pad(src, (0, epad - e_total)).reshape(epad // 128, 128)
    dst = jnp.pad(dst, (0, epad - e_total)).reshape(epad // 128, 128)

    feat, el, er = _tc_feat(in_feat, W, attn_l, attn_r)
    h_part, den_part = _sc_edge_pass(src, dst, feat, el, er, n, e_total, h)
    return _tc_final(h_part, den_part, bias, W2, b2)
